# baseline (speedup 1.0000x reference)
import functools

import jax
import jax.numpy as jnp
from jax import lax
from jax.experimental import pallas as pl
from jax.experimental.pallas import tpu as pltpu
from jax.experimental.pallas import tpu_sc as plsc

VOCAB = 100000
DIM = 64
B = 1024
T = 200
NGRAM_COUNT = 4 * T - 10
NCHUNK = 10
CHUNK = 80
IDS_PAD = NCHUNK * CHUNK

NC = 2
NS = 16
NW = NC * NS
ROWS_PER_W = B // NW

CHUNK_COUNTS = tuple(CHUNK if k < NCHUNK - 1 else NGRAM_COUNT - (NCHUNK - 1) * CHUNK
                     for k in range(NCHUNK))


_TBLK = 4000


def _ids_body(x_ref, emb_ref, out_ref, emb16_ref):
    @pl.when(pl.program_id(0) == 0)
    def _():
        xp = x_ref[...] + 1
        i2 = xp[:, 0:199] * 257 + xp[:, 1:200]
        i3 = (i2[:, 0:198] * 257 + xp[:, 2:200]) % VOCAB
        i4 = (i3[:, 0:197] * 257 + xp[:, 3:200]) % VOCAB
        i5 = (i4[:, 0:196] * 257 + xp[:, 4:200]) % VOCAB
        pad = jnp.zeros((B, IDS_PAD - NGRAM_COUNT), dtype=jnp.int32)
        ids_all = jnp.concatenate([i2, i3, i4, i5, pad], axis=1)
        out_ref[...] = ids_all.reshape(B, NCHUNK, CHUNK)

    emb16_ref[...] = emb_ref[...].astype(jnp.bfloat16)


def _compute_ids(x32, emb_weight):
    return pl.pallas_call(
        _ids_body,
        grid=(VOCAB // _TBLK,),
        in_specs=[
            pl.BlockSpec((B, T), lambda i: (jnp.int32(0), jnp.int32(0))),
            pl.BlockSpec((_TBLK, DIM), lambda i: (i.astype(jnp.int32), jnp.int32(0))),
        ],
        out_specs=[
            pl.BlockSpec((B, NCHUNK, CHUNK),
                         lambda i: (jnp.int32(0), jnp.int32(0), jnp.int32(0))),
            pl.BlockSpec((_TBLK, DIM), lambda i: (i.astype(jnp.int32), jnp.int32(0))),
        ],
        out_shape=[
            jax.ShapeDtypeStruct((B, NCHUNK, CHUNK), jnp.int32),
            jax.ShapeDtypeStruct((VOCAB, DIM), jnp.bfloat16),
        ],
    )(x32, emb_weight)


def _sc_body(ids_hbm, table_hbm, out_hbm, idx_v, bufs, acc_v, sems):
    wid = lax.axis_index("s") * NC + lax.axis_index("c")
    base = wid * ROWS_PER_W

    pltpu.sync_copy(ids_hbm.at[pl.ds(base, ROWS_PER_W)], idx_v)

    def fire(r, k, slab):
        pltpu.async_copy(
            table_hbm.at[idx_v.at[r, jnp.int32(k)]],
            bufs[slab],
            sems[slab],
        )

    def wait(slab):
        pltpu.make_async_copy(
            table_hbm.at[pl.ds(0, CHUNK)], bufs[slab], sems[slab]
        ).wait()

    fire(jnp.int32(0), 0, 0)
    fire(jnp.int32(0), 1, 1)

    def row_body(r, _):
        z = jnp.zeros((16,), jnp.float32)
        acc = (z, z, z, z)
        hi_mask = jnp.full((16,), -65536, jnp.int32)

        for k in range(NCHUNK):
            slab = k % 2
            wait(slab)
            buf = bufs[slab]

            def acc_body(i, carry):
                ae0, ao0, ae1, ao1 = carry
                w0 = plsc.bitcast(buf[i, pl.ds(0, 32)], jnp.int32)
                w1 = plsc.bitcast(buf[i, pl.ds(32, 32)], jnp.int32)
                ae0 = ae0 + plsc.bitcast(w0 << 16, jnp.float32)
                ao0 = ao0 + plsc.bitcast(w0 & hi_mask, jnp.float32)
                ae1 = ae1 + plsc.bitcast(w1 << 16, jnp.float32)
                ao1 = ao1 + plsc.bitcast(w1 & hi_mask, jnp.float32)
                return (ae0, ao0, ae1, ao1)

            acc = plsc.parallel_loop(
                jnp.int32(0), jnp.int32(CHUNK_COUNTS[k]), jnp.int32(1),
                unroll=5, carry=acc)(acc_body)

            if k < NCHUNK - 2:
                fire(r, k + 2, slab)
            else:

                @pl.when(r < ROWS_PER_W - 1)
                def _():
                    fire(r + 1, k + 2 - NCHUNK, slab)

        a0, a1, a2, a3 = acc
        acc_v[r, pl.ds(0, 16)] = a0
        acc_v[r, pl.ds(16, 16)] = a1
        acc_v[r, pl.ds(32, 16)] = a2
        acc_v[r, pl.ds(48, 16)] = a3
        return _

    lax.fori_loop(jnp.int32(0), jnp.int32(ROWS_PER_W), row_body, None)
    pltpu.sync_copy(acc_v, out_hbm.at[pl.ds(base, ROWS_PER_W)])


def _sc_entry(ids_hbm, table_hbm, out_hbm, idx_v, buf_a, buf_b, acc_v,
              sem_a, sem_b):
    _sc_body(ids_hbm, table_hbm, out_hbm, idx_v, (buf_a, buf_b), acc_v,
             (sem_a, sem_b))


@functools.cache
def _gather_sums_fn():
    return pl.kernel(
        _sc_entry,
        out_type=jax.ShapeDtypeStruct((B, DIM), jnp.float32),
        mesh=plsc.VectorSubcoreMesh(core_axis_name="c", subcore_axis_name="s"),
        scratch_types=[
            pltpu.VMEM((ROWS_PER_W, NCHUNK, CHUNK), jnp.int32),
            pltpu.VMEM((CHUNK, DIM), jnp.bfloat16),
            pltpu.VMEM((CHUNK, DIM), jnp.bfloat16),
            pltpu.VMEM((ROWS_PER_W, DIM), jnp.float32),
            pltpu.SemaphoreType.DMA,
            pltpu.SemaphoreType.DMA,
        ],
        compiler_params=pltpu.CompilerParams(
            use_tc_tiling_on_sc=False, needs_layout_passes=False),
    )


def _norm_body(s_ref, out_ref):
    p = s_ref[...] * (1.0 / NGRAM_COUNT)
    n2 = jnp.sum(p * p, axis=1, keepdims=True)
    norm = jnp.maximum(jnp.sqrt(n2), 1e-12)
    out_ref[...] = p / norm


def _normalize(sums):
    return pl.pallas_call(
        _norm_body,
        out_shape=jax.ShapeDtypeStruct((B, DIM), jnp.float32),
    )(sums)


_UNPERM = tuple(
    (j // 2 if j % 2 == 0 else 16 + j // 2) if j < 32
    else (32 + (j - 32) // 2 if j % 2 == 0 else 48 + (j - 32) // 2)
    for j in range(DIM)
)


def kernel(x_bytes, emb_weight):
    x32 = x_bytes.astype(jnp.int32)
    ids, emb16 = _compute_ids(x32, emb_weight)
    sums = _gather_sums_fn()(ids, emb16)
    out_perm = _normalize(sums)
    return jnp.take(out_perm, jnp.array(_UNPERM, jnp.int32), axis=1)


# speedup vs baseline: 1.1174x; 1.1174x over previous
import functools

import jax
import jax.numpy as jnp
from jax import lax
from jax.experimental import pallas as pl
from jax.experimental.pallas import tpu as pltpu
from jax.experimental.pallas import tpu_sc as plsc

VOCAB = 100000
DIM = 64
B = 1024
T = 200
NGRAM_COUNT = 4 * T - 10
NCHUNK = 10
CHUNK = 80
IDS_PAD = NCHUNK * CHUNK

NC = 2
NS = 16
NW = NC * NS
ROWS_PER_W = B // NW

CHUNK_COUNTS = tuple(CHUNK if k < NCHUNK - 1 else NGRAM_COUNT - (NCHUNK - 1) * CHUNK
                     for k in range(NCHUNK))


def _ids_body(x_ref, out_ref):
    xp = x_ref[...] + 1
    i2 = xp[:, 0:199] * 257 + xp[:, 1:200]
    i3 = (i2[:, 0:198] * 257 + xp[:, 2:200]) % VOCAB
    i4 = (i3[:, 0:197] * 257 + xp[:, 3:200]) % VOCAB
    i5 = (i4[:, 0:196] * 257 + xp[:, 4:200]) % VOCAB
    pad = jnp.zeros((B, IDS_PAD - NGRAM_COUNT), dtype=jnp.int32)
    ids_all = jnp.concatenate([i2, i3, i4, i5, pad], axis=1)
    out_ref[...] = ids_all.reshape(B, NCHUNK, CHUNK)


def _compute_ids(x32):
    return pl.pallas_call(
        _ids_body,
        out_shape=jax.ShapeDtypeStruct((B, NCHUNK, CHUNK), jnp.int32),
    )(x32)


_CROWS = 125
_ROWS_PER_CW = VOCAB // NW


def _conv_body(emb_hbm, out_hbm, fbuf, bbuf):
    wid = lax.axis_index("s") * NC + lax.axis_index("c")
    vbase = wid * _ROWS_PER_CW

    def step(s, _):
        row0 = vbase + s * _CROWS
        pltpu.sync_copy(emb_hbm.at[pl.ds(row0, _CROWS)], fbuf)

        def pack_row(i):
            f0 = fbuf[i, pl.ds(0, 16)]
            f1 = fbuf[i, pl.ds(16, 16)]
            f2 = fbuf[i, pl.ds(32, 16)]
            f3 = fbuf[i, pl.ds(48, 16)]
            bbuf[i, pl.ds(0, 32)] = plsc.pack(
                f0, f1, format=plsc.PackFormat.INTERLEAVED)
            bbuf[i, pl.ds(32, 32)] = plsc.pack(
                f2, f3, format=plsc.PackFormat.INTERLEAVED)

        plsc.parallel_loop(jnp.int32(0), jnp.int32(_CROWS), jnp.int32(1),
                           unroll=5)(pack_row)
        pltpu.sync_copy(bbuf, out_hbm.at[pl.ds(row0, _CROWS)])
        return _

    lax.fori_loop(jnp.int32(0), jnp.int32(_ROWS_PER_CW // _CROWS), step, None)


@functools.cache
def _convert_fn():
    return pl.kernel(
        _conv_body,
        out_type=jax.ShapeDtypeStruct((VOCAB, DIM), jnp.bfloat16),
        mesh=plsc.VectorSubcoreMesh(core_axis_name="c", subcore_axis_name="s"),
        scratch_types=[
            pltpu.VMEM((_CROWS, DIM), jnp.float32),
            pltpu.VMEM((_CROWS, DIM), jnp.bfloat16),
        ],
        compiler_params=pltpu.CompilerParams(
            use_tc_tiling_on_sc=False, needs_layout_passes=False),
    )


def _sc_body(ids_hbm, table_hbm, out_hbm, idx_v, bufs, acc_v, sems):
    wid = lax.axis_index("s") * NC + lax.axis_index("c")
    base = wid * ROWS_PER_W

    pltpu.sync_copy(ids_hbm.at[pl.ds(base, ROWS_PER_W)], idx_v)

    def fire(r, k, slab):
        pltpu.async_copy(
            table_hbm.at[idx_v.at[r, jnp.int32(k)]],
            bufs[slab],
            sems[slab],
        )

    def wait(slab):
        pltpu.make_async_copy(
            table_hbm.at[pl.ds(0, CHUNK)], bufs[slab], sems[slab]
        ).wait()

    fire(jnp.int32(0), 0, 0)
    fire(jnp.int32(0), 1, 1)

    def row_body(r, _):
        z = jnp.zeros((16,), jnp.float32)
        acc = (z, z, z, z)
        hi_mask = jnp.full((16,), -65536, jnp.int32)

        for k in range(NCHUNK):
            slab = k % 2
            wait(slab)
            buf = bufs[slab]

            def acc_body(i, carry):
                ae0, ao0, ae1, ao1 = carry
                w0 = plsc.bitcast(buf[i, pl.ds(0, 32)], jnp.int32)
                w1 = plsc.bitcast(buf[i, pl.ds(32, 32)], jnp.int32)
                ae0 = ae0 + plsc.bitcast(w0 << 16, jnp.float32)
                ao0 = ao0 + plsc.bitcast(w0 & hi_mask, jnp.float32)
                ae1 = ae1 + plsc.bitcast(w1 << 16, jnp.float32)
                ao1 = ao1 + plsc.bitcast(w1 & hi_mask, jnp.float32)
                return (ae0, ao0, ae1, ao1)

            acc = plsc.parallel_loop(
                jnp.int32(0), jnp.int32(CHUNK_COUNTS[k]), jnp.int32(1),
                unroll=5, carry=acc)(acc_body)

            if k < NCHUNK - 2:
                fire(r, k + 2, slab)
            else:

                @pl.when(r < ROWS_PER_W - 1)
                def _():
                    fire(r + 1, k + 2 - NCHUNK, slab)

        a0, a1, a2, a3 = acc
        acc_v[r, pl.ds(0, 16)] = a0
        acc_v[r, pl.ds(16, 16)] = a1
        acc_v[r, pl.ds(32, 16)] = a2
        acc_v[r, pl.ds(48, 16)] = a3
        return _

    lax.fori_loop(jnp.int32(0), jnp.int32(ROWS_PER_W), row_body, None)
    pltpu.sync_copy(acc_v, out_hbm.at[pl.ds(base, ROWS_PER_W)])


def _sc_entry(ids_hbm, table_hbm, out_hbm, idx_v, buf_a, buf_b, acc_v,
              sem_a, sem_b):
    _sc_body(ids_hbm, table_hbm, out_hbm, idx_v, (buf_a, buf_b), acc_v,
             (sem_a, sem_b))


@functools.cache
def _gather_sums_fn():
    return pl.kernel(
        _sc_entry,
        out_type=jax.ShapeDtypeStruct((B, DIM), jnp.float32),
        mesh=plsc.VectorSubcoreMesh(core_axis_name="c", subcore_axis_name="s"),
        scratch_types=[
            pltpu.VMEM((ROWS_PER_W, NCHUNK, CHUNK), jnp.int32),
            pltpu.VMEM((CHUNK, DIM), jnp.bfloat16),
            pltpu.VMEM((CHUNK, DIM), jnp.bfloat16),
            pltpu.VMEM((ROWS_PER_W, DIM), jnp.float32),
            pltpu.SemaphoreType.DMA,
            pltpu.SemaphoreType.DMA,
        ],
        compiler_params=pltpu.CompilerParams(
            use_tc_tiling_on_sc=False, needs_layout_passes=False),
    )


def _norm_body(s_ref, out_ref):
    p = s_ref[...] * (1.0 / NGRAM_COUNT)
    n2 = jnp.sum(p * p, axis=1, keepdims=True)
    norm = jnp.maximum(jnp.sqrt(n2), 1e-12)
    out_ref[...] = p / norm


def _normalize(sums):
    return pl.pallas_call(
        _norm_body,
        out_shape=jax.ShapeDtypeStruct((B, DIM), jnp.float32),
    )(sums)


def kernel(x_bytes, emb_weight):
    x32 = x_bytes.astype(jnp.int32)
    ids = _compute_ids(x32)
    emb16 = _convert_fn()(emb_weight)
    sums = _gather_sums_fn()(ids, emb16)
    return _normalize(sums)


# speedup vs baseline: 1.1906x; 1.0655x over previous
import functools

import jax
import jax.numpy as jnp
from jax import lax
from jax.experimental import pallas as pl
from jax.experimental.pallas import tpu as pltpu
from jax.experimental.pallas import tpu_sc as plsc

VOCAB = 100000
DIM = 64
B = 1024
T = 200
NGRAM_COUNT = 4 * T - 10
NCHUNK = 10
CHUNK = 80
IDS_PAD = NCHUNK * CHUNK

NC = 2
NS = 16
NW = NC * NS
ROWS_PER_W = B // NW

CHUNK_COUNTS = tuple(CHUNK if k < NCHUNK - 1 else NGRAM_COUNT - (NCHUNK - 1) * CHUNK
                     for k in range(NCHUNK))


def _ids_body(x_ref, out_ref):
    xp = x_ref[...] + 1
    i2 = xp[:, 0:199] * 257 + xp[:, 1:200]
    i3 = (i2[:, 0:198] * 257 + xp[:, 2:200]) % VOCAB
    i4 = (i3[:, 0:197] * 257 + xp[:, 3:200]) % VOCAB
    i5 = (i4[:, 0:196] * 257 + xp[:, 4:200]) % VOCAB
    pad = jnp.zeros((B, IDS_PAD - NGRAM_COUNT), dtype=jnp.int32)
    ids_all = jnp.concatenate([i2, i3, i4, i5, pad], axis=1)
    out_ref[...] = ids_all.reshape(B, NCHUNK, CHUNK)


def _compute_ids(x32):
    return pl.pallas_call(
        _ids_body,
        out_shape=jax.ShapeDtypeStruct((B, NCHUNK, CHUNK), jnp.int32),
    )(x32)


_CROWS = 200
_CCHUNKS = VOCAB // _CROWS


def _conv_body(embt_hbm, out_hbm, fbuf, bbuf):
    wid = lax.axis_index("s") * NC + lax.axis_index("c")
    iota = lax.iota(jnp.int32, 16)

    def step(j, _):
        c = wid + j * NW
        row0 = c * _CROWS
        pltpu.sync_copy(embt_hbm.at[:, pl.ds(row0, _CROWS)], fbuf)

        def pack_row(i):
            vsplat = jnp.zeros((16,), jnp.int32) + i
            f0 = plsc.load_gather(fbuf, [iota, vsplat])
            f1 = plsc.load_gather(fbuf, [iota + 16, vsplat])
            f2 = plsc.load_gather(fbuf, [iota + 32, vsplat])
            f3 = plsc.load_gather(fbuf, [iota + 48, vsplat])
            bbuf[i, pl.ds(0, 32)] = plsc.pack(
                f0, f1, format=plsc.PackFormat.INTERLEAVED)
            bbuf[i, pl.ds(32, 32)] = plsc.pack(
                f2, f3, format=plsc.PackFormat.INTERLEAVED)

        plsc.parallel_loop(jnp.int32(0), jnp.int32(_CROWS), jnp.int32(1),
                           unroll=4)(pack_row)
        pltpu.sync_copy(bbuf, out_hbm.at[pl.ds(row0, _CROWS)])
        return _

    nch = jnp.int32(_CCHUNKS // NW) + (
        wid < jnp.int32(_CCHUNKS % NW)).astype(jnp.int32)
    lax.fori_loop(jnp.int32(0), nch, step, None)


@functools.cache
def _convert_fn():
    return pl.kernel(
        _conv_body,
        out_type=jax.ShapeDtypeStruct((VOCAB, DIM), jnp.bfloat16),
        mesh=plsc.VectorSubcoreMesh(core_axis_name="c", subcore_axis_name="s"),
        scratch_types=[
            pltpu.VMEM((DIM, _CROWS), jnp.float32),
            pltpu.VMEM((_CROWS, DIM), jnp.bfloat16),
        ],
        compiler_params=pltpu.CompilerParams(
            use_tc_tiling_on_sc=False, needs_layout_passes=False),
    )


def _sc_body(ids_hbm, table_hbm, out_hbm, idx_v, bufs, acc_v, sems):
    wid = lax.axis_index("s") * NC + lax.axis_index("c")
    base = wid * ROWS_PER_W

    pltpu.sync_copy(ids_hbm.at[pl.ds(base, ROWS_PER_W)], idx_v)

    def fire(r, k, slab):
        pltpu.async_copy(
            table_hbm.at[idx_v.at[r, jnp.int32(k)]],
            bufs[slab],
            sems[slab],
        )

    def wait(slab):
        pltpu.make_async_copy(
            table_hbm.at[pl.ds(0, CHUNK)], bufs[slab], sems[slab]
        ).wait()

    fire(jnp.int32(0), 0, 0)
    fire(jnp.int32(0), 1, 1)

    def row_body(r, _):
        z = jnp.zeros((16,), jnp.float32)
        acc = (z, z, z, z)
        hi_mask = jnp.full((16,), -65536, jnp.int32)

        for k in range(NCHUNK):
            slab = k % 2
            wait(slab)
            buf = bufs[slab]

            def acc_body(i, carry):
                ae0, ao0, ae1, ao1 = carry
                w0 = plsc.bitcast(buf[i, pl.ds(0, 32)], jnp.int32)
                w1 = plsc.bitcast(buf[i, pl.ds(32, 32)], jnp.int32)
                ae0 = ae0 + plsc.bitcast(w0 << 16, jnp.float32)
                ao0 = ao0 + plsc.bitcast(w0 & hi_mask, jnp.float32)
                ae1 = ae1 + plsc.bitcast(w1 << 16, jnp.float32)
                ao1 = ao1 + plsc.bitcast(w1 & hi_mask, jnp.float32)
                return (ae0, ao0, ae1, ao1)

            acc = plsc.parallel_loop(
                jnp.int32(0), jnp.int32(CHUNK_COUNTS[k]), jnp.int32(1),
                unroll=5, carry=acc)(acc_body)

            if k < NCHUNK - 2:
                fire(r, k + 2, slab)
            else:

                @pl.when(r < ROWS_PER_W - 1)
                def _():
                    fire(r + 1, k + 2 - NCHUNK, slab)

        a0, a1, a2, a3 = acc
        acc_v[r, pl.ds(0, 16)] = a0
        acc_v[r, pl.ds(16, 16)] = a1
        acc_v[r, pl.ds(32, 16)] = a2
        acc_v[r, pl.ds(48, 16)] = a3
        return _

    lax.fori_loop(jnp.int32(0), jnp.int32(ROWS_PER_W), row_body, None)
    pltpu.sync_copy(acc_v, out_hbm.at[pl.ds(base, ROWS_PER_W)])


def _sc_entry(ids_hbm, table_hbm, out_hbm, idx_v, buf_a, buf_b, acc_v,
              sem_a, sem_b):
    _sc_body(ids_hbm, table_hbm, out_hbm, idx_v, (buf_a, buf_b), acc_v,
             (sem_a, sem_b))


@functools.cache
def _gather_sums_fn():
    return pl.kernel(
        _sc_entry,
        out_type=jax.ShapeDtypeStruct((B, DIM), jnp.float32),
        mesh=plsc.VectorSubcoreMesh(core_axis_name="c", subcore_axis_name="s"),
        scratch_types=[
            pltpu.VMEM((ROWS_PER_W, NCHUNK, CHUNK), jnp.int32),
            pltpu.VMEM((CHUNK, DIM), jnp.bfloat16),
            pltpu.VMEM((CHUNK, DIM), jnp.bfloat16),
            pltpu.VMEM((ROWS_PER_W, DIM), jnp.float32),
            pltpu.SemaphoreType.DMA,
            pltpu.SemaphoreType.DMA,
        ],
        compiler_params=pltpu.CompilerParams(
            use_tc_tiling_on_sc=False, needs_layout_passes=False),
    )


def _norm_body(s_ref, out_ref):
    p = s_ref[...] * (1.0 / NGRAM_COUNT)
    n2 = jnp.sum(p * p, axis=1, keepdims=True)
    norm = jnp.maximum(jnp.sqrt(n2), 1e-12)
    out_ref[...] = p / norm


def _normalize(sums):
    return pl.pallas_call(
        _norm_body,
        out_shape=jax.ShapeDtypeStruct((B, DIM), jnp.float32),
    )(sums)


def kernel(x_bytes, emb_weight):
    x32 = x_bytes.astype(jnp.int32)
    ids = _compute_ids(x32)
    emb16 = _convert_fn()(emb_weight.T)
    sums = _gather_sums_fn()(ids, emb16)
    return _normalize(sums)


# speedup vs baseline: 1.2126x; 1.0185x over previous
import functools

import jax
import jax.numpy as jnp
from jax import lax
from jax.experimental import pallas as pl
from jax.experimental.pallas import tpu as pltpu
from jax.experimental.pallas import tpu_sc as plsc

VOCAB = 100000
DIM = 64
B = 1024
T = 200
NGRAM_COUNT = 4 * T - 10
NCHUNK = 10
CHUNK = 80
IDS_PAD = NCHUNK * CHUNK

NC = 2
NS = 16
NW = NC * NS
ROWS_PER_W = B // NW

CHUNK_COUNTS = tuple(CHUNK if k < NCHUNK - 1 else NGRAM_COUNT - (NCHUNK - 1) * CHUNK
                     for k in range(NCHUNK))


def _ids_body(x_ref, out_ref):
    xp = x_ref[...] + 1
    i2 = xp[:, 0:199] * 257 + xp[:, 1:200]
    i3 = (i2[:, 0:198] * 257 + xp[:, 2:200]) % VOCAB
    i4 = (i3[:, 0:197] * 257 + xp[:, 3:200]) % VOCAB
    i5 = (i4[:, 0:196] * 257 + xp[:, 4:200]) % VOCAB
    pad = jnp.zeros((B, IDS_PAD - NGRAM_COUNT), dtype=jnp.int32)
    ids_all = jnp.concatenate([i2, i3, i4, i5, pad], axis=1)
    out_ref[...] = ids_all.reshape(B, NCHUNK, CHUNK)


def _compute_ids(x32):
    return pl.pallas_call(
        _ids_body,
        out_shape=jax.ShapeDtypeStruct((B, NCHUNK, CHUNK), jnp.int32),
    )(x32)


_CROWS = 200
_CCHUNKS = VOCAB // _CROWS


def _conv_body(embt_hbm, out_hbm, fbuf, bbuf):
    wid = lax.axis_index("s") * NC + lax.axis_index("c")
    iota = lax.iota(jnp.int32, 16)

    def step(j, _):
        c = wid + j * NW
        row0 = c * _CROWS
        pltpu.sync_copy(embt_hbm.at[:, pl.ds(row0, _CROWS)], fbuf)

        def pack_row(i):
            vsplat = jnp.zeros((16,), jnp.int32) + i
            f0 = plsc.load_gather(fbuf, [iota, vsplat])
            f1 = plsc.load_gather(fbuf, [iota + 16, vsplat])
            f2 = plsc.load_gather(fbuf, [iota + 32, vsplat])
            f3 = plsc.load_gather(fbuf, [iota + 48, vsplat])
            bbuf[i, pl.ds(0, 32)] = plsc.pack(
                f0, f1, format=plsc.PackFormat.INTERLEAVED)
            bbuf[i, pl.ds(32, 32)] = plsc.pack(
                f2, f3, format=plsc.PackFormat.INTERLEAVED)

        plsc.parallel_loop(jnp.int32(0), jnp.int32(_CROWS), jnp.int32(1),
                           unroll=4)(pack_row)
        pltpu.sync_copy(bbuf, out_hbm.at[pl.ds(row0, _CROWS)])
        return _

    nch = jnp.int32(_CCHUNKS // NW) + (
        wid < jnp.int32(_CCHUNKS % NW)).astype(jnp.int32)
    lax.fori_loop(jnp.int32(0), nch, step, None)


@functools.cache
def _convert_fn():
    return pl.kernel(
        _conv_body,
        out_type=jax.ShapeDtypeStruct((VOCAB, DIM), jnp.bfloat16),
        mesh=plsc.VectorSubcoreMesh(core_axis_name="c", subcore_axis_name="s"),
        scratch_types=[
            pltpu.VMEM((DIM, _CROWS), jnp.float32),
            pltpu.VMEM((_CROWS, DIM), jnp.bfloat16),
        ],
        compiler_params=pltpu.CompilerParams(
            use_tc_tiling_on_sc=False, needs_layout_passes=False),
    )


def _sc_body(ids_hbm, table_hbm, out_hbm, idx_v, bufs, acc_v, sems):
    wid = lax.axis_index("s") * NC + lax.axis_index("c")
    base = wid * ROWS_PER_W

    pltpu.sync_copy(ids_hbm.at[pl.ds(base, ROWS_PER_W)], idx_v)

    def fire(r, k, slab):
        pltpu.async_copy(
            table_hbm.at[idx_v.at[r, jnp.int32(k)]],
            bufs[slab],
            sems[slab],
        )

    def wait(slab):
        pltpu.make_async_copy(
            table_hbm.at[pl.ds(0, CHUNK)], bufs[slab], sems[slab]
        ).wait()

    fire(jnp.int32(0), 0, 0)
    fire(jnp.int32(0), 1, 1)

    def row_body(r, _):
        z = jnp.zeros((16,), jnp.float32)
        acc = (z, z, z, z)
        hi_mask = jnp.full((16,), -65536, jnp.int32)

        for k in range(NCHUNK):
            slab = k % 2
            wait(slab)
            buf = bufs[slab]

            def acc_body(i, carry):
                ae0, ao0, ae1, ao1 = carry
                w0 = plsc.bitcast(buf[i, pl.ds(0, 32)], jnp.int32)
                w1 = plsc.bitcast(buf[i, pl.ds(32, 32)], jnp.int32)
                ae0 = ae0 + plsc.bitcast(w0 << 16, jnp.float32)
                ao0 = ao0 + plsc.bitcast(w0 & hi_mask, jnp.float32)
                ae1 = ae1 + plsc.bitcast(w1 << 16, jnp.float32)
                ao1 = ao1 + plsc.bitcast(w1 & hi_mask, jnp.float32)
                return (ae0, ao0, ae1, ao1)

            acc = plsc.parallel_loop(
                jnp.int32(0), jnp.int32(CHUNK_COUNTS[k]), jnp.int32(1),
                unroll=5, carry=acc)(acc_body)

            if k < NCHUNK - 2:
                fire(r, k + 2, slab)
            else:

                @pl.when(r < ROWS_PER_W - 1)
                def _():
                    fire(r + 1, k + 2 - NCHUNK, slab)

        a0, a1, a2, a3 = acc
        sq = a0 * a0 + a1 * a1 + a2 * a2 + a3 * a3
        n2 = lax.reduce_sum_p.bind(sq, axes=(0,))
        n2v = jnp.maximum(jnp.zeros((16,), jnp.float32) + n2,
                          jnp.float32(6.241e-19))
        u = plsc.bitcast(n2v, jnp.int32)
        y = plsc.bitcast(jnp.int32(0x5F3759DF) - (u >> 1), jnp.float32)
        for _i in range(3):
            y = y * (1.5 - 0.5 * n2v * y * y)
        acc_v[r, pl.ds(0, 16)] = a0 * y
        acc_v[r, pl.ds(16, 16)] = a1 * y
        acc_v[r, pl.ds(32, 16)] = a2 * y
        acc_v[r, pl.ds(48, 16)] = a3 * y
        return _

    lax.fori_loop(jnp.int32(0), jnp.int32(ROWS_PER_W), row_body, None)
    pltpu.sync_copy(acc_v, out_hbm.at[pl.ds(base, ROWS_PER_W)])


def _sc_entry(ids_hbm, table_hbm, out_hbm, idx_v, buf_a, buf_b, acc_v,
              sem_a, sem_b):
    _sc_body(ids_hbm, table_hbm, out_hbm, idx_v, (buf_a, buf_b), acc_v,
             (sem_a, sem_b))


@functools.cache
def _gather_sums_fn():
    return pl.kernel(
        _sc_entry,
        out_type=jax.ShapeDtypeStruct((B, DIM), jnp.float32),
        mesh=plsc.VectorSubcoreMesh(core_axis_name="c", subcore_axis_name="s"),
        scratch_types=[
            pltpu.VMEM((ROWS_PER_W, NCHUNK, CHUNK), jnp.int32),
            pltpu.VMEM((CHUNK, DIM), jnp.bfloat16),
            pltpu.VMEM((CHUNK, DIM), jnp.bfloat16),
            pltpu.VMEM((ROWS_PER_W, DIM), jnp.float32),
            pltpu.SemaphoreType.DMA,
            pltpu.SemaphoreType.DMA,
        ],
        compiler_params=pltpu.CompilerParams(
            use_tc_tiling_on_sc=False, needs_layout_passes=False),
    )


def _norm_body(s_ref, out_ref):
    p = s_ref[...] * (1.0 / NGRAM_COUNT)
    n2 = jnp.sum(p * p, axis=1, keepdims=True)
    norm = jnp.maximum(jnp.sqrt(n2), 1e-12)
    out_ref[...] = p / norm


def _normalize(sums):
    return pl.pallas_call(
        _norm_body,
        out_shape=jax.ShapeDtypeStruct((B, DIM), jnp.float32),
    )(sums)


def kernel(x_bytes, emb_weight):
    x32 = x_bytes.astype(jnp.int32)
    ids = _compute_ids(x32)
    emb16 = _convert_fn()(emb_weight.T)
    return _gather_sums_fn()(ids, emb16)
